# Initial kernel scaffold; baseline (speedup 1.0000x reference)
import jax
import jax.numpy as jnp
from jax.experimental import pallas as pl


def kernel(anchors, rpn_bbox_pred, scores):
    raise NotImplementedError("write your pallas kernel here")


# speedup vs baseline: 17.6058x; 17.6058x over previous
import functools

import jax
import jax.numpy as jnp
from jax import lax
from jax.experimental import pallas as pl
from jax.experimental.pallas import tpu as pltpu
from jax.experimental.pallas import tpu_sc as plsc

N = 20000
NT = 16
TN = 1280
NP = NT * TN
C = TN // 16
MAX_OUT = 128
IOU_T = 0.7
IM_HI = 223.0
NEG = float("-inf")
BIG = 1e9


def _spl(x):
    return jnp.broadcast_to(x, (16,))


def _nms_body(a_hbm, d_hbm, s_hbm, out_hbm,
              ax1, ay1, az1, ax2, ay2, az2,
              dxr, dyr, dzr, dwr, dhr, dlr,
              vols, sc, recbuf, recs, outbuf, shared):
    sid = lax.axis_index("s")
    base = sid * TN
    base_f = base.astype(jnp.float32)

    for i, r in enumerate((ax1, ay1, az1, ax2, ay2, az2)):
        pltpu.sync_copy(a_hbm.at[i, pl.ds(base, TN)], r)
    for i, r in enumerate((dxr, dyr, dzr, dwr, dhr, dlr)):
        pltpu.sync_copy(d_hbm.at[i, pl.ds(base, TN)], r)
    pltpu.sync_copy(s_hbm.at[pl.ds(base, TN)], sc)

    lane_i = lax.iota(jnp.int32, 16)
    lane_f = lane_i.astype(jnp.float32)

    def decode_chunk(k, carry):
        vmax, vidx = carry
        sl = pl.ds(k * 16, 16)
        x1a = ax1[sl]; y1a = ay1[sl]; z1a = az1[sl]
        x2a = ax2[sl]; y2a = ay2[sl]; z2a = az2[sl]
        dx = dxr[sl]; dy = dyr[sl]; dz = dzr[sl]
        dw = dwr[sl]; dh = dhr[sl]; dl = dlr[sl]
        w = x2a - x1a + 1.0
        h = y2a - y1a + 1.0
        ln = z2a - z1a + 1.0
        cx = x1a + w * 0.5
        cy = y1a + h * 0.5
        cz = z1a + ln * 0.5
        pcx = dx * w + cx
        pcy = dy * h + cy
        pcz = dz * ln + cz
        pw = jnp.exp(dw) * w
        ph = jnp.exp(dh) * h
        plen = jnp.exp(dl) * ln
        x1 = jnp.clip(pcx - pw * 0.5, 0.0, IM_HI)
        y1 = jnp.clip(pcy - ph * 0.5, 0.0, IM_HI)
        z1 = jnp.clip(pcz - plen * 0.5, 0.0, IM_HI)
        x2 = jnp.clip(pcx + pw * 0.5, 0.0, IM_HI)
        y2 = jnp.clip(pcy + ph * 0.5, 0.0, IM_HI)
        z2 = jnp.clip(pcz + plen * 0.5, 0.0, IM_HI)
        ax1[sl] = x1; ay1[sl] = y1; az1[sl] = z1
        ax2[sl] = x2; ay2[sl] = y2; az2[sl] = z2
        vols[sl] = (x2 - x1 + 1.0) * (y2 - y1 + 1.0) * (z2 - z1 + 1.0)
        ss = sc[sl]
        idxv = _spl(base_f + (k * 16).astype(jnp.float32)) + lane_f
        pred = ss > vmax
        return jnp.where(pred, ss, vmax), jnp.where(pred, idxv, vidx)

    init = (_spl(jnp.float32(NEG)), _spl(jnp.float32(BIG)))
    vmax, vidx = lax.fori_loop(0, C, decode_chunk, init)

    def step(t, carry):
        vmax, vidx = carry
        m = jnp.max(vmax)
        li = jnp.min(jnp.where(vmax == m, vidx, BIG))
        lvalid = m > jnp.float32(-1e30)
        loc = jnp.where(lvalid, li - base_f, 0.0).astype(jnp.int32)
        loci = _spl(loc)
        rec = jnp.zeros((16,), jnp.float32)
        for ci, r in enumerate((ax1, ay1, az1, ax2, ay2, az2)):
            rec = jnp.where(lane_i == ci, plsc.load_gather(r, [loci]), rec)
        rec = jnp.where(lane_i == 6, _spl(m), rec)
        rec = jnp.where(lane_i == 7, _spl(li), rec)
        recbuf[...] = rec
        par = (t % 2) * 256
        pltpu.sync_copy(recbuf, shared.at[pl.ds(par + sid * 16, 16)])
        plsc.subcore_barrier()
        pltpu.sync_copy(shared.at[pl.ds(par, 256)], recs)
        s_r = plsc.load_gather(recs, [lane_i * 16 + 6])
        i_r = plsc.load_gather(recs, [lane_i * 16 + 7])
        gm = jnp.max(s_r)
        gmask = s_r == gm
        gi = jnp.min(jnp.where(gmask, i_r, BIG))
        valid = gm > jnp.float32(-1e30)
        rwin = jnp.min(jnp.where(gmask & (i_r == gi), lane_i, 16))
        rwin = jnp.where(rwin > 15, 0, rwin)
        rbase = rwin * 16
        rec16 = plsc.load_gather(recs, [_spl(rbase) + lane_i])
        validv = _spl(valid)
        row = jnp.where((lane_i < 7) & validv, rec16, 0.0)
        outbuf[pl.ds(t * 16, 16)] = row
        cx1 = plsc.load_gather(recs, [_spl(rbase + 0)])
        cy1 = plsc.load_gather(recs, [_spl(rbase + 1)])
        cz1 = plsc.load_gather(recs, [_spl(rbase + 2)])
        cx2 = plsc.load_gather(recs, [_spl(rbase + 3)])
        cy2 = plsc.load_gather(recs, [_spl(rbase + 4)])
        cz2 = plsc.load_gather(recs, [_spl(rbase + 5)])
        vol0 = (cx2 - cx1 + 1.0) * (cy2 - cy1 + 1.0) * (cz2 - cz1 + 1.0)

        def supp_chunk(k, c2):
            nvmax, nvidx = c2
            sl = pl.ds(k * 16, 16)
            x1 = ax1[sl]; y1 = ay1[sl]; z1 = az1[sl]
            x2 = ax2[sl]; y2 = ay2[sl]; z2 = az2[sl]
            vv = vols[sl]; ss = sc[sl]
            xx1 = jnp.maximum(cx1, x1)
            yy1 = jnp.maximum(cy1, y1)
            zz1 = jnp.maximum(cz1, z1)
            xx2 = jnp.minimum(cx2, x2)
            yy2 = jnp.minimum(cy2, y2)
            zz2 = jnp.minimum(cz2, z2)
            iw = jnp.maximum(xx2 - xx1 + 1.0, 0.0)
            ih = jnp.maximum(yy2 - yy1 + 1.0, 0.0)
            il = jnp.maximum(zz2 - zz1 + 1.0, 0.0)
            inter = iw * ih * il
            iou = inter / (vol0 + vv - inter)
            ns = jnp.where((iou >= IOU_T) & validv, NEG, ss)
            sc[sl] = ns
            idxv = _spl(base_f + (k * 16).astype(jnp.float32)) + lane_f
            pred = ns > nvmax
            return jnp.where(pred, ns, nvmax), jnp.where(pred, idxv, nvidx)

        return lax.fori_loop(0, C, supp_chunk, init)

    lax.fori_loop(0, MAX_OUT, step, (vmax, vidx))

    @pl.when(sid == 0)
    def _():
        pltpu.sync_copy(outbuf, out_hbm)


_scratch = (
    [pltpu.VMEM((TN,), jnp.float32) for _ in range(12)]
    + [pltpu.VMEM((TN,), jnp.float32),
       pltpu.VMEM((TN,), jnp.float32),
       pltpu.VMEM((16,), jnp.float32),
       pltpu.VMEM((256,), jnp.float32),
       pltpu.VMEM((MAX_OUT * 16,), jnp.float32),
       pltpu.VMEM_SHARED((512,), jnp.float32)]
)

_nms_call = pl.kernel(
    _nms_body,
    out_type=jax.ShapeDtypeStruct((MAX_OUT * 16,), jnp.float32),
    mesh=plsc.VectorSubcoreMesh(core_axis_name="c", subcore_axis_name="s",
                                num_cores=1, num_subcores=NT),
    scratch_types=_scratch,
    compiler_params=pltpu.CompilerParams(needs_layout_passes=False),
)


def kernel(anchors, rpn_bbox_pred, scores):
    pad = NP - N
    a_t = jnp.pad(jnp.transpose(anchors.astype(jnp.float32)), ((0, 0), (0, pad)))
    d_t = jnp.pad(jnp.transpose(rpn_bbox_pred.astype(jnp.float32)), ((0, 0), (0, pad)))
    s_p = jnp.pad(scores.astype(jnp.float32), (0, pad), constant_values=NEG)
    out = _nms_call(a_t, d_t, s_p)
    return out.reshape(MAX_OUT, 16)[:, :7]


# speedup vs baseline: 17.8658x; 1.0148x over previous
import functools

import jax
import jax.numpy as jnp
from jax import lax
from jax.experimental import pallas as pl
from jax.experimental.pallas import tpu as pltpu
from jax.experimental.pallas import tpu_sc as plsc

N = 20000
NT = 16
TN = 1280
NP = NT * TN
C = TN // 16
MAX_OUT = 128
IOU_T = 0.7
IM_HI = 223.0
NEG = float("-inf")
BIG = 1e9


def _spl(x):
    return jnp.broadcast_to(x, (16,))


def _nms_body(a_hbm, d_hbm, s_hbm, out_hbm,
              ax1, ay1, az1, ax2, ay2, az2,
              dxr, dyr, dzr, dwr, dhr, dlr,
              vols, sc, recbuf, recs, outbuf, shared):
    sid = lax.axis_index("s")
    base = sid * TN
    base_f = base.astype(jnp.float32)

    for i, r in enumerate((ax1, ay1, az1, ax2, ay2, az2)):
        pltpu.sync_copy(a_hbm.at[i, pl.ds(base, TN)], r)
    for i, r in enumerate((dxr, dyr, dzr, dwr, dhr, dlr)):
        pltpu.sync_copy(d_hbm.at[i, pl.ds(base, TN)], r)
    pltpu.sync_copy(s_hbm.at[pl.ds(base, TN)], sc)

    lane_i = lax.iota(jnp.int32, 16)
    lane_f = lane_i.astype(jnp.float32)

    def decode_chunk(k, carry):
        vmax, vidx = carry
        sl = pl.ds(k * 16, 16)
        x1a = ax1[sl]; y1a = ay1[sl]; z1a = az1[sl]
        x2a = ax2[sl]; y2a = ay2[sl]; z2a = az2[sl]
        dx = dxr[sl]; dy = dyr[sl]; dz = dzr[sl]
        dw = dwr[sl]; dh = dhr[sl]; dl = dlr[sl]
        w = x2a - x1a + 1.0
        h = y2a - y1a + 1.0
        ln = z2a - z1a + 1.0
        cx = x1a + w * 0.5
        cy = y1a + h * 0.5
        cz = z1a + ln * 0.5
        pcx = dx * w + cx
        pcy = dy * h + cy
        pcz = dz * ln + cz
        pw = jnp.exp(dw) * w
        ph = jnp.exp(dh) * h
        plen = jnp.exp(dl) * ln
        x1 = jnp.clip(pcx - pw * 0.5, 0.0, IM_HI)
        y1 = jnp.clip(pcy - ph * 0.5, 0.0, IM_HI)
        z1 = jnp.clip(pcz - plen * 0.5, 0.0, IM_HI)
        x2 = jnp.clip(pcx + pw * 0.5, 0.0, IM_HI)
        y2 = jnp.clip(pcy + ph * 0.5, 0.0, IM_HI)
        z2 = jnp.clip(pcz + plen * 0.5, 0.0, IM_HI)
        ax1[sl] = x1; ay1[sl] = y1; az1[sl] = z1
        ax2[sl] = x2; ay2[sl] = y2; az2[sl] = z2
        vols[sl] = (x2 - x1 + 1.0) * (y2 - y1 + 1.0) * (z2 - z1 + 1.0)
        ss = sc[sl]
        idxv = _spl(base_f + (k * 16).astype(jnp.float32)) + lane_f
        pred = ss > vmax
        return jnp.where(pred, ss, vmax), jnp.where(pred, idxv, vidx)

    init = (_spl(jnp.float32(NEG)), _spl(jnp.float32(BIG)))
    vmax, vidx = lax.fori_loop(0, C, decode_chunk, init)

    def step(t, carry):
        vmax, vidx = carry
        m = jnp.max(vmax)
        li = jnp.min(jnp.where(vmax == m, vidx, BIG))
        lvalid = m > jnp.float32(-1e30)
        loc = jnp.where(lvalid, li - base_f, 0.0).astype(jnp.int32)
        loci = _spl(loc)
        rec = jnp.zeros((16,), jnp.float32)
        for ci, r in enumerate((ax1, ay1, az1, ax2, ay2, az2)):
            rec = jnp.where(lane_i == ci, plsc.load_gather(r, [loci]), rec)
        rec = jnp.where(lane_i == 6, _spl(m), rec)
        rec = jnp.where(lane_i == 7, _spl(li), rec)
        recbuf[...] = rec
        par = (t % 2) * 256
        pltpu.sync_copy(recbuf, shared.at[pl.ds(par + sid * 16, 16)])
        plsc.subcore_barrier()
        pltpu.sync_copy(shared.at[pl.ds(par, 256)], recs)
        s_r = plsc.load_gather(recs, [lane_i * 16 + 6])
        i_r = plsc.load_gather(recs, [lane_i * 16 + 7])
        gm = jnp.max(s_r)
        gmask = s_r == gm
        gi = jnp.min(jnp.where(gmask, i_r, BIG))
        valid = gm > jnp.float32(-1e30)
        rwin = jnp.min(jnp.where(gmask & (i_r == gi), lane_i, 16))
        rwin = jnp.where(rwin > 15, 0, rwin)
        rbase = rwin * 16
        rec16 = plsc.load_gather(recs, [_spl(rbase) + lane_i])
        validv = _spl(valid)
        row = jnp.where((lane_i < 7) & validv, rec16, 0.0)
        outbuf[pl.ds(t * 16, 16)] = row
        cx1 = plsc.load_gather(recs, [_spl(rbase + 0)])
        cy1 = plsc.load_gather(recs, [_spl(rbase + 1)])
        cz1 = plsc.load_gather(recs, [_spl(rbase + 2)])
        cx2 = plsc.load_gather(recs, [_spl(rbase + 3)])
        cy2 = plsc.load_gather(recs, [_spl(rbase + 4)])
        cz2 = plsc.load_gather(recs, [_spl(rbase + 5)])
        vol0 = (cx2 - cx1 + 1.0) * (cy2 - cy1 + 1.0) * (cz2 - cz1 + 1.0)

        def supp_chunk(k4, c2):
            nvmax, nvidx = c2
            for u in range(4):
                k = k4 * 4 + u
                sl = pl.ds(k * 16, 16)
                x1 = ax1[sl]; y1 = ay1[sl]; z1 = az1[sl]
                x2 = ax2[sl]; y2 = ay2[sl]; z2 = az2[sl]
                vv = vols[sl]; ss = sc[sl]
                xx1 = jnp.maximum(cx1, x1)
                yy1 = jnp.maximum(cy1, y1)
                zz1 = jnp.maximum(cz1, z1)
                xx2 = jnp.minimum(cx2, x2)
                yy2 = jnp.minimum(cy2, y2)
                zz2 = jnp.minimum(cz2, z2)
                iw = jnp.maximum(xx2 - xx1 + 1.0, 0.0)
                ih = jnp.maximum(yy2 - yy1 + 1.0, 0.0)
                il = jnp.maximum(zz2 - zz1 + 1.0, 0.0)
                inter = iw * ih * il
                iou = inter / (vol0 + vv - inter)
                ns = jnp.where((iou >= IOU_T) & validv, NEG, ss)
                sc[sl] = ns
                idxv = _spl(base_f + (k * 16).astype(jnp.float32)) + lane_f
                pred = ns > nvmax
                nvmax = jnp.where(pred, ns, nvmax)
                nvidx = jnp.where(pred, idxv, nvidx)
            return nvmax, nvidx

        return lax.fori_loop(0, C // 4, supp_chunk, init)

    lax.fori_loop(0, MAX_OUT, step, (vmax, vidx))

    @pl.when(sid == 0)
    def _():
        pltpu.sync_copy(outbuf, out_hbm)


_scratch = (
    [pltpu.VMEM((TN,), jnp.float32) for _ in range(12)]
    + [pltpu.VMEM((TN,), jnp.float32),
       pltpu.VMEM((TN,), jnp.float32),
       pltpu.VMEM((16,), jnp.float32),
       pltpu.VMEM((256,), jnp.float32),
       pltpu.VMEM((MAX_OUT * 16,), jnp.float32),
       pltpu.VMEM_SHARED((512,), jnp.float32)]
)

_nms_call = pl.kernel(
    _nms_body,
    out_type=jax.ShapeDtypeStruct((MAX_OUT * 16,), jnp.float32),
    mesh=plsc.VectorSubcoreMesh(core_axis_name="c", subcore_axis_name="s",
                                num_cores=1, num_subcores=NT),
    scratch_types=_scratch,
    compiler_params=pltpu.CompilerParams(needs_layout_passes=False),
)


def kernel(anchors, rpn_bbox_pred, scores):
    pad = NP - N
    a_t = jnp.pad(jnp.transpose(anchors.astype(jnp.float32)), ((0, 0), (0, pad)))
    d_t = jnp.pad(jnp.transpose(rpn_bbox_pred.astype(jnp.float32)), ((0, 0), (0, pad)))
    s_p = jnp.pad(scores.astype(jnp.float32), (0, pad), constant_values=NEG)
    out = _nms_call(a_t, d_t, s_p)
    return out.reshape(MAX_OUT, 16)[:, :7]


# speedup vs baseline: 18.3307x; 1.0260x over previous
import functools

import jax
import jax.numpy as jnp
from jax import lax
from jax.experimental import pallas as pl
from jax.experimental.pallas import tpu as pltpu
from jax.experimental.pallas import tpu_sc as plsc

N = 20000
NT = 16
TN = 1280
NP = NT * TN
C = TN // 16
MAX_OUT = 128
IOU_T = 0.7
IM_HI = 223.0
NEG = float("-inf")
BIG = 1e9


def _spl(x):
    return jnp.broadcast_to(x, (16,))


def _nms_body(a_hbm, d_hbm, s_hbm, out_hbm,
              ax1, ay1, az1, ax2, ay2, az2,
              dxr, dyr, dzr, dwr, dhr, dlr,
              vols, sc, recbuf, recs, outbuf, shared):
    sid = lax.axis_index("s")
    base = sid * TN
    base_f = base.astype(jnp.float32)

    for i, r in enumerate((ax1, ay1, az1, ax2, ay2, az2)):
        pltpu.sync_copy(a_hbm.at[i, pl.ds(base, TN)], r)
    for i, r in enumerate((dxr, dyr, dzr, dwr, dhr, dlr)):
        pltpu.sync_copy(d_hbm.at[i, pl.ds(base, TN)], r)
    pltpu.sync_copy(s_hbm.at[pl.ds(base, TN)], sc)

    lane_i = lax.iota(jnp.int32, 16)
    lane_f = lane_i.astype(jnp.float32)

    def decode_chunk(k, carry):
        vmax, vidx = carry
        sl = pl.ds(k * 16, 16)
        x1a = ax1[sl]; y1a = ay1[sl]; z1a = az1[sl]
        x2a = ax2[sl]; y2a = ay2[sl]; z2a = az2[sl]
        dx = dxr[sl]; dy = dyr[sl]; dz = dzr[sl]
        dw = dwr[sl]; dh = dhr[sl]; dl = dlr[sl]
        w = x2a - x1a + 1.0
        h = y2a - y1a + 1.0
        ln = z2a - z1a + 1.0
        cx = x1a + w * 0.5
        cy = y1a + h * 0.5
        cz = z1a + ln * 0.5
        pcx = dx * w + cx
        pcy = dy * h + cy
        pcz = dz * ln + cz
        pw = jnp.exp(dw) * w
        ph = jnp.exp(dh) * h
        plen = jnp.exp(dl) * ln
        x1 = jnp.clip(pcx - pw * 0.5, 0.0, IM_HI)
        y1 = jnp.clip(pcy - ph * 0.5, 0.0, IM_HI)
        z1 = jnp.clip(pcz - plen * 0.5, 0.0, IM_HI)
        x2 = jnp.clip(pcx + pw * 0.5, 0.0, IM_HI)
        y2 = jnp.clip(pcy + ph * 0.5, 0.0, IM_HI)
        z2 = jnp.clip(pcz + plen * 0.5, 0.0, IM_HI)
        ax1[sl] = x1; ay1[sl] = y1; az1[sl] = z1
        ax2[sl] = x2; ay2[sl] = y2; az2[sl] = z2
        vols[sl] = (x2 - x1 + 1.0) * (y2 - y1 + 1.0) * (z2 - z1 + 1.0)
        ss = sc[sl]
        idxv = _spl(base_f + (k * 16).astype(jnp.float32)) + lane_f
        pred = ss > vmax
        return jnp.where(pred, ss, vmax), jnp.where(pred, idxv, vidx)

    init = (_spl(jnp.float32(NEG)), _spl(jnp.float32(BIG)))
    vmax, vidx = lax.fori_loop(0, C, decode_chunk, init)

    def step(t, carry):
        vmax, vidx = carry
        m = jnp.max(vmax)
        li = jnp.min(jnp.where(vmax == m, vidx, BIG))
        lvalid = m > jnp.float32(-1e30)
        loc = jnp.where(lvalid, li - base_f, 0.0).astype(jnp.int32)
        loci = _spl(loc)
        rec = jnp.zeros((16,), jnp.float32)
        for ci, r in enumerate((ax1, ay1, az1, ax2, ay2, az2)):
            rec = jnp.where(lane_i == ci, plsc.load_gather(r, [loci]), rec)
        rec = jnp.where(lane_i == 6, _spl(m), rec)
        rec = jnp.where(lane_i == 7, _spl(li), rec)
        recbuf[...] = rec
        par = (t % 2) * 128
        pltpu.sync_copy(recbuf.at[pl.ds(0, 8)],
                        shared.at[pl.ds(par + sid * 8, 8)])
        plsc.subcore_barrier()
        pltpu.sync_copy(shared.at[pl.ds(par, 128)], recs)
        s_r = plsc.load_gather(recs, [lane_i * 8 + 6])
        i_r = plsc.load_gather(recs, [lane_i * 8 + 7])
        gm = jnp.max(s_r)
        gmask = s_r == gm
        gi = jnp.min(jnp.where(gmask, i_r, BIG))
        valid = gm > jnp.float32(-1e30)
        rbase = _spl(plsc.all_reduce_ffs(gmask & (i_r == gi))) * 8
        rec16 = plsc.load_gather(recs, [rbase + jnp.minimum(lane_i, 7)])
        validv = _spl(valid)
        row = jnp.where((lane_i < 7) & validv, rec16, 0.0)
        outbuf[pl.ds(t * 16, 16)] = row
        cx1 = plsc.load_gather(recs, [rbase + 0])
        cy1 = plsc.load_gather(recs, [rbase + 1])
        cz1 = plsc.load_gather(recs, [rbase + 2])
        cx2 = plsc.load_gather(recs, [rbase + 3])
        cy2 = plsc.load_gather(recs, [rbase + 4])
        cz2 = plsc.load_gather(recs, [rbase + 5])
        vol0 = (cx2 - cx1 + 1.0) * (cy2 - cy1 + 1.0) * (cz2 - cz1 + 1.0)

        def supp_chunk(k4, c2):
            nvmax, nvidx = c2
            for u in range(4):
                k = k4 * 4 + u
                sl = pl.ds(k * 16, 16)
                x1 = ax1[sl]; y1 = ay1[sl]; z1 = az1[sl]
                x2 = ax2[sl]; y2 = ay2[sl]; z2 = az2[sl]
                vv = vols[sl]; ss = sc[sl]
                xx1 = jnp.maximum(cx1, x1)
                yy1 = jnp.maximum(cy1, y1)
                zz1 = jnp.maximum(cz1, z1)
                xx2 = jnp.minimum(cx2, x2)
                yy2 = jnp.minimum(cy2, y2)
                zz2 = jnp.minimum(cz2, z2)
                iw = jnp.maximum(xx2 - xx1 + 1.0, 0.0)
                ih = jnp.maximum(yy2 - yy1 + 1.0, 0.0)
                il = jnp.maximum(zz2 - zz1 + 1.0, 0.0)
                inter = iw * ih * il
                iou = inter / (vol0 + vv - inter)
                ns = jnp.where((iou >= IOU_T) & validv, NEG, ss)
                sc[sl] = ns
                idxv = _spl(base_f + (k * 16).astype(jnp.float32)) + lane_f
                pred = ns > nvmax
                nvmax = jnp.where(pred, ns, nvmax)
                nvidx = jnp.where(pred, idxv, nvidx)
            return nvmax, nvidx

        return lax.fori_loop(0, C // 4, supp_chunk, init)

    lax.fori_loop(0, MAX_OUT, step, (vmax, vidx))

    @pl.when(sid == 0)
    def _():
        pltpu.sync_copy(outbuf, out_hbm)


_scratch = (
    [pltpu.VMEM((TN,), jnp.float32) for _ in range(12)]
    + [pltpu.VMEM((TN,), jnp.float32),
       pltpu.VMEM((TN,), jnp.float32),
       pltpu.VMEM((16,), jnp.float32),
       pltpu.VMEM((128,), jnp.float32),
       pltpu.VMEM((MAX_OUT * 16,), jnp.float32),
       pltpu.VMEM_SHARED((256,), jnp.float32)]
)

_nms_call = pl.kernel(
    _nms_body,
    out_type=jax.ShapeDtypeStruct((MAX_OUT * 16,), jnp.float32),
    mesh=plsc.VectorSubcoreMesh(core_axis_name="c", subcore_axis_name="s",
                                num_cores=1, num_subcores=NT),
    scratch_types=_scratch,
    compiler_params=pltpu.CompilerParams(needs_layout_passes=False),
)


def kernel(anchors, rpn_bbox_pred, scores):
    pad = NP - N
    a_t = jnp.pad(jnp.transpose(anchors.astype(jnp.float32)), ((0, 0), (0, pad)))
    d_t = jnp.pad(jnp.transpose(rpn_bbox_pred.astype(jnp.float32)), ((0, 0), (0, pad)))
    s_p = jnp.pad(scores.astype(jnp.float32), (0, pad), constant_values=NEG)
    out = _nms_call(a_t, d_t, s_p)
    return out.reshape(MAX_OUT, 16)[:, :7]


# speedup vs baseline: 18.7389x; 1.0223x over previous
import functools

import jax
import jax.numpy as jnp
from jax import lax
from jax.experimental import pallas as pl
from jax.experimental.pallas import tpu as pltpu
from jax.experimental.pallas import tpu_sc as plsc

N = 20000
NT = 16
TN = 1280
NP = NT * TN
C = TN // 16
MAX_OUT = 128
IOU_T = 0.7
IM_HI = 223.0
NEG = float("-inf")
BIG = 1e9


def _spl(x):
    return jnp.broadcast_to(x, (16,))


def _nms_body(a_hbm, d_hbm, s_hbm, out_hbm,
              ax1, ay1, az1, ax2, ay2, az2,
              dxr, dyr, dzr, dwr, dhr, dlr,
              vols, sc, recbuf, recs, outbuf, shared):
    sid = lax.axis_index("s")
    base = sid * TN
    base_f = base.astype(jnp.float32)

    for i, r in enumerate((ax1, ay1, az1, ax2, ay2, az2)):
        pltpu.sync_copy(a_hbm.at[i, pl.ds(base, TN)], r)
    for i, r in enumerate((dxr, dyr, dzr, dwr, dhr, dlr)):
        pltpu.sync_copy(d_hbm.at[i, pl.ds(base, TN)], r)
    pltpu.sync_copy(s_hbm.at[pl.ds(base, TN)], sc)

    lane_i = lax.iota(jnp.int32, 16)
    lane_f = lane_i.astype(jnp.float32)

    def decode_chunk(k, carry):
        vmax, vidx = carry
        sl = pl.ds(k * 16, 16)
        x1a = ax1[sl]; y1a = ay1[sl]; z1a = az1[sl]
        x2a = ax2[sl]; y2a = ay2[sl]; z2a = az2[sl]
        dx = dxr[sl]; dy = dyr[sl]; dz = dzr[sl]
        dw = dwr[sl]; dh = dhr[sl]; dl = dlr[sl]
        w = x2a - x1a + 1.0
        h = y2a - y1a + 1.0
        ln = z2a - z1a + 1.0
        cx = x1a + w * 0.5
        cy = y1a + h * 0.5
        cz = z1a + ln * 0.5
        pcx = dx * w + cx
        pcy = dy * h + cy
        pcz = dz * ln + cz
        pw = jnp.exp(dw) * w
        ph = jnp.exp(dh) * h
        plen = jnp.exp(dl) * ln
        x1 = jnp.clip(pcx - pw * 0.5, 0.0, IM_HI)
        y1 = jnp.clip(pcy - ph * 0.5, 0.0, IM_HI)
        z1 = jnp.clip(pcz - plen * 0.5, 0.0, IM_HI)
        x2 = jnp.clip(pcx + pw * 0.5, 0.0, IM_HI)
        y2 = jnp.clip(pcy + ph * 0.5, 0.0, IM_HI)
        z2 = jnp.clip(pcz + plen * 0.5, 0.0, IM_HI)
        ax1[sl] = x1; ay1[sl] = y1; az1[sl] = z1
        ax2[sl] = x2; ay2[sl] = y2; az2[sl] = z2
        vols[sl] = (x2 - x1 + 1.0) * (y2 - y1 + 1.0) * (z2 - z1 + 1.0)
        ss = sc[sl]
        idxv = _spl(base_f + (k * 16).astype(jnp.float32)) + lane_f
        pred = ss > vmax
        return jnp.where(pred, ss, vmax), jnp.where(pred, idxv, vidx)

    init = (_spl(jnp.float32(NEG)), _spl(jnp.float32(BIG)))
    vmax, vidx = lax.fori_loop(0, C, decode_chunk, init)

    def step(t, carry):
        vmax, vidx = carry
        m = jnp.max(vmax)
        li = jnp.min(jnp.where(vmax == m, vidx, BIG))
        lvalid = m > jnp.float32(-1e30)
        loc = jnp.where(lvalid, li - base_f, 0.0).astype(jnp.int32)
        loci = _spl(loc)
        rec = jnp.zeros((16,), jnp.float32)
        for ci, r in enumerate((ax1, ay1, az1, ax2, ay2, az2)):
            rec = jnp.where(lane_i == ci, plsc.load_gather(r, [loci]), rec)
        rec = jnp.where(lane_i == 6, _spl(m), rec)
        rec = jnp.where(lane_i == 7, _spl(li), rec)
        recbuf[...] = rec
        par = (t % 2) * 128
        pltpu.sync_copy(recbuf.at[pl.ds(0, 8)],
                        shared.at[pl.ds(par + sid * 8, 8)])
        plsc.subcore_barrier()
        pltpu.sync_copy(shared.at[pl.ds(par, 128)], recs)
        s_r = plsc.load_gather(recs, [lane_i * 8 + 6])
        i_r = plsc.load_gather(recs, [lane_i * 8 + 7])
        gm = jnp.max(s_r)
        gmask = s_r == gm
        gi = jnp.min(jnp.where(gmask, i_r, BIG))
        valid = gm > jnp.float32(-1e30)
        rbase = _spl(plsc.all_reduce_ffs(gmask & (i_r == gi))) * 8
        rec16 = plsc.load_gather(recs, [rbase + jnp.minimum(lane_i, 7)])
        validv = _spl(valid)
        row = jnp.where((lane_i < 7) & validv, rec16, 0.0)
        outbuf[pl.ds(t * 16, 16)] = row
        cx1 = plsc.load_gather(recs, [rbase + 0])
        cx1 = jnp.where(validv, cx1, jnp.float32(jnp.inf))
        cy1 = plsc.load_gather(recs, [rbase + 1])
        cz1 = plsc.load_gather(recs, [rbase + 2])
        cx2 = plsc.load_gather(recs, [rbase + 3])
        cy2 = plsc.load_gather(recs, [rbase + 4])
        cz2 = plsc.load_gather(recs, [rbase + 5])
        vol0 = (cx2 - cx1 + 1.0) * (cy2 - cy1 + 1.0) * (cz2 - cz1 + 1.0)

        def supp_chunk(k4, c2):
            nvmax, nvidx = c2
            for u in range(4):
                k = k4 * 4 + u
                sl = pl.ds(k * 16, 16)
                x1 = ax1[sl]; y1 = ay1[sl]; z1 = az1[sl]
                x2 = ax2[sl]; y2 = ay2[sl]; z2 = az2[sl]
                vv = vols[sl]; ss = sc[sl]
                xx1 = jnp.maximum(cx1, x1)
                yy1 = jnp.maximum(cy1, y1)
                zz1 = jnp.maximum(cz1, z1)
                xx2 = jnp.minimum(cx2, x2)
                yy2 = jnp.minimum(cy2, y2)
                zz2 = jnp.minimum(cz2, z2)
                iw = jnp.maximum(xx2 - xx1 + 1.0, 0.0)
                ih = jnp.maximum(yy2 - yy1 + 1.0, 0.0)
                il = jnp.maximum(zz2 - zz1 + 1.0, 0.0)
                inter = iw * ih * il
                iou = inter / (vol0 + vv - inter)
                ns = jnp.where(iou >= IOU_T, NEG, ss)
                sc[sl] = ns
                idxv = _spl(base_f + (k * 16).astype(jnp.float32)) + lane_f
                pred = ns > nvmax
                nvmax = jnp.where(pred, ns, nvmax)
                nvidx = jnp.where(pred, idxv, nvidx)
            return nvmax, nvidx

        return lax.fori_loop(0, C // 4, supp_chunk, init)

    lax.fori_loop(0, MAX_OUT, step, (vmax, vidx))

    @pl.when(sid == 0)
    def _():
        pltpu.sync_copy(outbuf, out_hbm)


_scratch = (
    [pltpu.VMEM((TN,), jnp.float32) for _ in range(12)]
    + [pltpu.VMEM((TN,), jnp.float32),
       pltpu.VMEM((TN,), jnp.float32),
       pltpu.VMEM((16,), jnp.float32),
       pltpu.VMEM((128,), jnp.float32),
       pltpu.VMEM((MAX_OUT * 16,), jnp.float32),
       pltpu.VMEM_SHARED((256,), jnp.float32)]
)

_nms_call = pl.kernel(
    _nms_body,
    out_type=jax.ShapeDtypeStruct((MAX_OUT * 16,), jnp.float32),
    mesh=plsc.VectorSubcoreMesh(core_axis_name="c", subcore_axis_name="s",
                                num_cores=1, num_subcores=NT),
    scratch_types=_scratch,
    compiler_params=pltpu.CompilerParams(needs_layout_passes=False),
)


def kernel(anchors, rpn_bbox_pred, scores):
    pad = NP - N
    a_t = jnp.pad(jnp.transpose(anchors.astype(jnp.float32)), ((0, 0), (0, pad)))
    d_t = jnp.pad(jnp.transpose(rpn_bbox_pred.astype(jnp.float32)), ((0, 0), (0, pad)))
    s_p = jnp.pad(scores.astype(jnp.float32), (0, pad), constant_values=NEG)
    out = _nms_call(a_t, d_t, s_p)
    return out.reshape(MAX_OUT, 16)[:, :7]


# speedup vs baseline: 18.7903x; 1.0027x over previous
import functools

import jax
import jax.numpy as jnp
from jax import lax
from jax.experimental import pallas as pl
from jax.experimental.pallas import tpu as pltpu
from jax.experimental.pallas import tpu_sc as plsc

N = 20000
NT = 16
TN = 1280
NP = NT * TN
C = TN // 16
MAX_OUT = 128
IOU_T = 0.7
IM_HI = 223.0
NEG = float("-inf")
BIG = 1e9


def _spl(x):
    return jnp.broadcast_to(x, (16,))


def _nms_body(a_hbm, d_hbm, s_hbm, out_hbm,
              ax1, ay1, az1, ax2, ay2, az2,
              dxr, dyr, dzr, dwr, dhr, dlr,
              vols, sc, recbuf, recs, outbuf, shared):
    sid = lax.axis_index("s")
    base = sid * TN
    base_f = base.astype(jnp.float32)

    for i, r in enumerate((ax1, ay1, az1, ax2, ay2, az2)):
        pltpu.sync_copy(a_hbm.at[i, pl.ds(base, TN)], r)
    for i, r in enumerate((dxr, dyr, dzr, dwr, dhr, dlr)):
        pltpu.sync_copy(d_hbm.at[i, pl.ds(base, TN)], r)
    pltpu.sync_copy(s_hbm.at[pl.ds(base, TN)], sc)

    lane_i = lax.iota(jnp.int32, 16)
    lane_f = lane_i.astype(jnp.float32)

    def decode_chunk(k, carry):
        vmax, vidx = carry
        sl = pl.ds(k * 16, 16)
        x1a = ax1[sl]; y1a = ay1[sl]; z1a = az1[sl]
        x2a = ax2[sl]; y2a = ay2[sl]; z2a = az2[sl]
        dx = dxr[sl]; dy = dyr[sl]; dz = dzr[sl]
        dw = dwr[sl]; dh = dhr[sl]; dl = dlr[sl]
        w = x2a - x1a + 1.0
        h = y2a - y1a + 1.0
        ln = z2a - z1a + 1.0
        cx = x1a + w * 0.5
        cy = y1a + h * 0.5
        cz = z1a + ln * 0.5
        pcx = dx * w + cx
        pcy = dy * h + cy
        pcz = dz * ln + cz
        pw = jnp.exp(dw) * w
        ph = jnp.exp(dh) * h
        plen = jnp.exp(dl) * ln
        x1 = jnp.clip(pcx - pw * 0.5, 0.0, IM_HI)
        y1 = jnp.clip(pcy - ph * 0.5, 0.0, IM_HI)
        z1 = jnp.clip(pcz - plen * 0.5, 0.0, IM_HI)
        x2 = jnp.clip(pcx + pw * 0.5, 0.0, IM_HI)
        y2 = jnp.clip(pcy + ph * 0.5, 0.0, IM_HI)
        z2 = jnp.clip(pcz + plen * 0.5, 0.0, IM_HI)
        ax1[sl] = x1; ay1[sl] = y1; az1[sl] = z1
        ax2[sl] = x2; ay2[sl] = y2; az2[sl] = z2
        vols[sl] = (x2 - x1 + 1.0) * (y2 - y1 + 1.0) * (z2 - z1 + 1.0)
        ss = sc[sl]
        idxv = _spl(base_f + (k * 16).astype(jnp.float32)) + lane_f
        pred = ss > vmax
        return jnp.where(pred, ss, vmax), jnp.where(pred, idxv, vidx)

    init = (_spl(jnp.float32(NEG)), _spl(jnp.float32(BIG)))
    vmax, vidx = lax.fori_loop(0, C, decode_chunk, init)

    def step(t, carry):
        vmax, vidx = carry
        m = jnp.max(vmax)
        li = jnp.min(jnp.where(vmax == m, vidx, BIG))
        lvalid = m > jnp.float32(-1e30)
        loc = jnp.where(lvalid, li - base_f, 0.0).astype(jnp.int32)
        loci = _spl(loc)
        rec = jnp.zeros((16,), jnp.float32)
        for ci, r in enumerate((ax1, ay1, az1, ax2, ay2, az2)):
            rec = jnp.where(lane_i == ci, plsc.load_gather(r, [loci]), rec)
        rec = jnp.where(lane_i == 6, _spl(m), rec)
        rec = jnp.where(lane_i == 7, _spl(li), rec)
        recbuf[...] = rec
        par = (t % 2) * 128
        pltpu.sync_copy(recbuf.at[pl.ds(0, 8)],
                        shared.at[pl.ds(par + sid * 8, 8)])
        plsc.subcore_barrier()
        pltpu.sync_copy(shared.at[pl.ds(par, 128)], recs)
        s_r = plsc.load_gather(recs, [lane_i * 8 + 6])
        i_r = plsc.load_gather(recs, [lane_i * 8 + 7])
        gm = jnp.max(s_r)
        gmask = s_r == gm
        gi = jnp.min(jnp.where(gmask, i_r, BIG))
        valid = gm > jnp.float32(-1e30)
        rbase = _spl(plsc.all_reduce_ffs(gmask & (i_r == gi))) * 8
        rec16 = plsc.load_gather(recs, [rbase + jnp.minimum(lane_i, 7)])
        validv = _spl(valid)
        row = jnp.where((lane_i < 7) & validv, rec16, 0.0)
        outbuf[pl.ds(t * 16, 16)] = row
        cx1 = plsc.load_gather(recs, [rbase + 0])
        cx1 = jnp.where(validv, cx1, jnp.float32(jnp.inf))
        cy1 = plsc.load_gather(recs, [rbase + 1])
        cz1 = plsc.load_gather(recs, [rbase + 2])
        cx2 = plsc.load_gather(recs, [rbase + 3])
        cy2 = plsc.load_gather(recs, [rbase + 4])
        cz2 = plsc.load_gather(recs, [rbase + 5])
        vol0 = (cx2 - cx1 + 1.0) * (cy2 - cy1 + 1.0) * (cz2 - cz1 + 1.0)

        def supp_chunk(k4, c2):
            nvmax, nvidx = c2
            for u in range(4):
                k = k4 * 4 + u
                sl = pl.ds(k * 16, 16)
                x1 = ax1[sl]; y1 = ay1[sl]; z1 = az1[sl]
                x2 = ax2[sl]; y2 = ay2[sl]; z2 = az2[sl]
                vv = vols[sl]; ss = sc[sl]
                xx1 = jnp.maximum(cx1, x1)
                yy1 = jnp.maximum(cy1, y1)
                zz1 = jnp.maximum(cz1, z1)
                xx2 = jnp.minimum(cx2, x2)
                yy2 = jnp.minimum(cy2, y2)
                zz2 = jnp.minimum(cz2, z2)
                iw = jnp.maximum(xx2 - xx1 + 1.0, 0.0)
                ih = jnp.maximum(yy2 - yy1 + 1.0, 0.0)
                il = jnp.maximum(zz2 - zz1 + 1.0, 0.0)
                inter = iw * ih * il
                iou = inter / (vol0 + vv - inter)
                ns = jnp.where(iou >= IOU_T, NEG, ss)
                sc[sl] = ns
                idxv = _spl(base_f + (k * 16).astype(jnp.float32)) + lane_f
                pred = ns > nvmax
                nvmax = jnp.where(pred, ns, nvmax)
                nvidx = jnp.where(pred, idxv, nvidx)
            return nvmax, nvidx

        return plsc.parallel_loop(0, C // 4, carry=init)(supp_chunk)

    lax.fori_loop(0, MAX_OUT, step, (vmax, vidx))

    @pl.when(sid == 0)
    def _():
        pltpu.sync_copy(outbuf, out_hbm)


_scratch = (
    [pltpu.VMEM((TN,), jnp.float32) for _ in range(12)]
    + [pltpu.VMEM((TN,), jnp.float32),
       pltpu.VMEM((TN,), jnp.float32),
       pltpu.VMEM((16,), jnp.float32),
       pltpu.VMEM((128,), jnp.float32),
       pltpu.VMEM((MAX_OUT * 16,), jnp.float32),
       pltpu.VMEM_SHARED((256,), jnp.float32)]
)

_nms_call = pl.kernel(
    _nms_body,
    out_type=jax.ShapeDtypeStruct((MAX_OUT * 16,), jnp.float32),
    mesh=plsc.VectorSubcoreMesh(core_axis_name="c", subcore_axis_name="s",
                                num_cores=1, num_subcores=NT),
    scratch_types=_scratch,
    compiler_params=pltpu.CompilerParams(needs_layout_passes=False),
)


def kernel(anchors, rpn_bbox_pred, scores):
    pad = NP - N
    a_t = jnp.pad(jnp.transpose(anchors.astype(jnp.float32)), ((0, 0), (0, pad)))
    d_t = jnp.pad(jnp.transpose(rpn_bbox_pred.astype(jnp.float32)), ((0, 0), (0, pad)))
    s_p = jnp.pad(scores.astype(jnp.float32), (0, pad), constant_values=NEG)
    out = _nms_call(a_t, d_t, s_p)
    return out.reshape(MAX_OUT, 16)[:, :7]
